# Initial kernel scaffold; baseline (speedup 1.0000x reference)
import jax
import jax.numpy as jnp
from jax.experimental import pallas as pl


def kernel(x, edge_index, e, W_pre, b_pre, Wf1, bf1, Ws1, bs1, We1, be1, Wf2, bf2, Ws2, bs2, We2, be2, W_out, b_out):
    raise NotImplementedError("write your pallas kernel here")


# speedup vs baseline: 1.7738x; 1.7738x over previous
import functools

import jax
import jax.numpy as jnp
from jax import lax
from jax.experimental import pallas as pl
from jax.experimental.pallas import tpu as pltpu
from jax.experimental.pallas import tpu_sc as plsc

N = 10000
E = 160000
F = 128
DE = 16
DPE = 768

NC = 2
NS = 16
NW = NC * NS
PER_W = E // NW
C = 40
G_ITERS = PER_W // C
N_PAD = 10240
RT = N_PAD // NS

_SC_MESH = plsc.VectorSubcoreMesh(core_axis_name="c", subcore_axis_name="s")


def _gather_body(tab_hbm, dst_hbm, src_hbm, xd_hbm, xs_hbm,
                 idxd_v, idxs_v, rowd_v, rows_v, semd, sems):
    c = lax.axis_index("c")
    s = lax.axis_index("s")
    base = (c * NS + s) * PER_W

    def body(i, carry):
        off = pl.multiple_of(base + i * C, 8)
        pltpu.sync_copy(dst_hbm.at[pl.ds(off, C)], idxd_v)
        pltpu.sync_copy(src_hbm.at[pl.ds(off, C)], idxs_v)
        cpd = pltpu.async_copy(tab_hbm.at[idxd_v], rowd_v, semd)
        cps = pltpu.async_copy(tab_hbm.at[idxs_v], rows_v, sems)
        cpd.wait()
        cps.wait()
        pltpu.sync_copy(rowd_v, xd_hbm.at[pl.ds(off, C)])
        pltpu.sync_copy(rows_v, xs_hbm.at[pl.ds(off, C)])
        return carry

    lax.fori_loop(0, G_ITERS, body, 0)


_gather2 = functools.partial(
    pl.kernel,
    out_type=[jax.ShapeDtypeStruct((E, F), jnp.float32),
              jax.ShapeDtypeStruct((E, F), jnp.float32)],
    mesh=_SC_MESH,
    scratch_types=[pltpu.VMEM((C,), jnp.int32),
                   pltpu.VMEM((C,), jnp.int32),
                   pltpu.VMEM((C, F), jnp.float32),
                   pltpu.VMEM((C, F), jnp.float32),
                   pltpu.SemaphoreType.DMA,
                   pltpu.SemaphoreType.DMA],
)(_gather_body)


def _scatter_body(msg_hbm, x_hbm, zeros_hbm, dst_hbm, p0_hbm, p1_hbm,
                  idx_v, rows_v, acc_sh):
    c = lax.axis_index("c")
    s = lax.axis_index("s")

    roff = s * RT

    @pl.when(c == 0)
    def _():
        pltpu.sync_copy(x_hbm.at[pl.ds(roff, RT)], acc_sh.at[pl.ds(roff, RT)])

    @pl.when(c != 0)
    def _():
        pltpu.sync_copy(zeros_hbm.at[pl.ds(roff, RT)],
                        acc_sh.at[pl.ds(roff, RT)])

    plsc.subcore_barrier()

    base = (c * NS + s) * PER_W

    def body(i, carry):
        off = pl.multiple_of(base + i * C, 8)
        pltpu.sync_copy(dst_hbm.at[pl.ds(off, C)], idx_v)
        pltpu.sync_copy(msg_hbm.at[pl.ds(off, C)], rows_v)
        pltpu.sync_copy(rows_v, acc_sh.at[idx_v], add=True)
        return carry

    lax.fori_loop(0, G_ITERS, body, 0)

    plsc.subcore_barrier()

    @pl.when(c == 0)
    def _():
        pltpu.sync_copy(acc_sh.at[pl.ds(roff, RT)], p0_hbm.at[pl.ds(roff, RT)])

    @pl.when(c != 0)
    def _():
        pltpu.sync_copy(acc_sh.at[pl.ds(roff, RT)], p1_hbm.at[pl.ds(roff, RT)])


_scatter = functools.partial(
    pl.kernel,
    out_type=[jax.ShapeDtypeStruct((N_PAD, F), jnp.float32),
              jax.ShapeDtypeStruct((N_PAD, F), jnp.float32)],
    mesh=_SC_MESH,
    scratch_types=[pltpu.VMEM((C,), jnp.int32),
                   pltpu.VMEM((C, F), jnp.float32),
                   pltpu.VMEM_SHARED((N_PAD, F), jnp.float32)],
)(_scatter_body)


BE1 = 800
BE2 = 1000
BN = 1280


def _p1_body(e_ref, xd_ref, xs_ref, Wpre_ref, bpre_ref,
             Ws1_ref, bs1_ref, Wf1_ref, bf1_ref, We1_ref, be1_ref,
             Ws2e_ref, Wf2e_ref,
             msg_ref, c_ref, d_ref):
    e0 = jnp.tanh(
        jnp.dot(e_ref[...], Wpre_ref[...], preferred_element_type=jnp.float32)
        + bpre_ref[...])
    xd = xd_ref[...]
    xs = xs_ref[...]

    def mm3(W):
        return (jnp.dot(xd, W[:F], preferred_element_type=jnp.float32)
                + jnp.dot(xs, W[F:2 * F], preferred_element_type=jnp.float32)
                + jnp.dot(e0, W[2 * F:], preferred_element_type=jnp.float32))

    s1 = mm3(Ws1_ref[...]) + bs1_ref[...]
    f1 = mm3(Wf1_ref[...]) + bf1_ref[...]
    msg_ref[...] = jax.nn.relu(s1) * jax.nn.sigmoid(f1)
    g1 = jax.nn.sigmoid(mm3(We1_ref[...]) + be1_ref[...])
    e1 = e0 * (1.0 + g1)
    c_ref[...] = jnp.dot(e1, Ws2e_ref[...], preferred_element_type=jnp.float32)
    d_ref[...] = jnp.dot(e1, Wf2e_ref[...], preferred_element_type=jnp.float32)


def _p1(e, xd, xs, W_pre, b_pre, Ws1, bs1, Wf1, bf1, We1, be1, Ws2e, Wf2e):
    full = lambda shp: pl.BlockSpec(shp, lambda i: (0, 0))
    eb = lambda w: pl.BlockSpec((BE1, w), lambda i: (i, 0))
    return pl.pallas_call(
        _p1_body,
        grid=(E // BE1,),
        in_specs=[eb(DE), eb(F), eb(F),
                  full((DE, DPE)), full((1, DPE)),
                  full((2 * F + DPE, F)), full((1, F)),
                  full((2 * F + DPE, F)), full((1, F)),
                  full((2 * F + DPE, DPE)), full((1, DPE)),
                  full((DPE, F)), full((DPE, F))],
        out_specs=[eb(F), eb(F), eb(F)],
        out_shape=[jax.ShapeDtypeStruct((E, F), jnp.float32)] * 3,
    )(e, xd, xs, W_pre, b_pre, Ws1, bs1, Wf1, bf1, We1, be1, Ws2e, Wf2e)


def _add_body(p0_ref, p1_ref, x1_ref, sum_ref):
    i = pl.program_id(0)
    v = p0_ref[...] + p1_ref[...]
    x1_ref[...] = v

    @pl.when(i == 0)
    def _():
        sum_ref[...] = jnp.zeros_like(sum_ref)

    sum_ref[...] += jnp.sum(v, axis=0, keepdims=True)


def _addsum(p0, p1):
    return pl.pallas_call(
        _add_body,
        grid=(N_PAD // BN,),
        in_specs=[pl.BlockSpec((BN, F), lambda i: (i, 0)),
                  pl.BlockSpec((BN, F), lambda i: (i, 0))],
        out_specs=[pl.BlockSpec((BN, F), lambda i: (i, 0)),
                   pl.BlockSpec((1, F), lambda i: (0, 0))],
        out_shape=[jax.ShapeDtypeStruct((N_PAD, F), jnp.float32),
                   jax.ShapeDtypeStruct((1, F), jnp.float32)],
    )(p0, p1)


def _p2_body(x1d_ref, x1s_ref, c_ref, d_ref,
             Ws2n_ref, bs2_ref, Wf2n_ref, bf2_ref,
             sumx1_ref, Wout_ref, bout_ref, out_ref, acc_ref):
    i = pl.program_id(0)

    @pl.when(i == 0)
    def _():
        acc_ref[...] = jnp.zeros_like(acc_ref)

    x1d = x1d_ref[...]
    x1s = x1s_ref[...]
    Ws2n = Ws2n_ref[...]
    Wf2n = Wf2n_ref[...]
    s2 = (jnp.dot(x1d, Ws2n[:F], preferred_element_type=jnp.float32)
          + jnp.dot(x1s, Ws2n[F:], preferred_element_type=jnp.float32)
          + c_ref[...] + bs2_ref[...])
    f2 = (jnp.dot(x1d, Wf2n[:F], preferred_element_type=jnp.float32)
          + jnp.dot(x1s, Wf2n[F:], preferred_element_type=jnp.float32)
          + d_ref[...] + bf2_ref[...])
    m = jax.nn.relu(s2) * jax.nn.sigmoid(f2)
    acc_ref[...] += jnp.sum(m, axis=0, keepdims=True)

    @pl.when(i == pl.num_programs(0) - 1)
    def _():
        pooled = acc_ref[...] + sumx1_ref[...]
        logits = (jnp.dot(pooled, Wout_ref[...],
                          preferred_element_type=jnp.float32) + bout_ref[...])
        mx = jnp.max(logits, axis=-1, keepdims=True)
        ex = jnp.exp(logits - mx)
        out_ref[...] = ex / jnp.sum(ex, axis=-1, keepdims=True)


def _p2(x1d, x1s, cc, dd, Ws2n, bs2, Wf2n, bf2, sumx1, W_out, b_out):
    full = lambda shp: pl.BlockSpec(shp, lambda i: (0, 0))
    eb = lambda w: pl.BlockSpec((BE2, w), lambda i: (i, 0))
    return pl.pallas_call(
        _p2_body,
        grid=(E // BE2,),
        in_specs=[eb(F), eb(F), eb(F), eb(F),
                  full((2 * F, F)), full((1, F)),
                  full((2 * F, F)), full((1, F)),
                  full((1, F)), full((F, 32)), full((1, 32))],
        out_specs=full((1, 32)),
        out_shape=jax.ShapeDtypeStruct((1, 32), jnp.float32),
        scratch_shapes=[pltpu.VMEM((1, F), jnp.float32)],
    )(x1d, x1s, cc, dd, Ws2n, bs2, Wf2n, bf2, sumx1, W_out, b_out)


def kernel(x, edge_index, e,
           W_pre, b_pre,
           Wf1, bf1, Ws1, bs1, We1, be1,
           Wf2, bf2, Ws2, bs2, We2, be2,
           W_out, b_out):
    dst = edge_index[0]
    src = edge_index[1]
    zeros = jnp.zeros((N_PAD, F), jnp.float32)
    x_pad = jnp.pad(x, ((0, N_PAD - N), (0, 0)))

    xd, xs = _gather2(x, dst, src)
    msg1, cc, dd = _p1(e, xd, xs,
                       W_pre, b_pre.reshape(1, DPE),
                       Ws1, bs1.reshape(1, F),
                       Wf1, bf1.reshape(1, F),
                       We1, be1.reshape(1, DPE),
                       Ws2[2 * F:], Wf2[2 * F:])
    p0, p1 = _scatter(msg1, x_pad, zeros, dst)
    x1, sumx1 = _addsum(p0, p1)
    x1d, x1s = _gather2(x1, dst, src)
    out = _p2(x1d, x1s, cc, dd,
              Ws2[:2 * F], bs2.reshape(1, F),
              Wf2[:2 * F], bf2.reshape(1, F),
              sumx1, W_out, b_out.reshape(1, 32))
    return out.reshape(32)


# speedup vs baseline: 2.3306x; 1.3139x over previous
import functools

import jax
import jax.numpy as jnp
from jax import lax
from jax.experimental import pallas as pl
from jax.experimental.pallas import tpu as pltpu
from jax.experimental.pallas import tpu_sc as plsc

N = 10000
E = 160000
F = 128
DE = 16
DPE = 768

NC = 2
NS = 16
NW = NC * NS
PER_W = E // NW
C = 40
G_ITERS = PER_W // C
N_PAD = 10240
RT = N_PAD // NS

_SC_MESH = plsc.VectorSubcoreMesh(core_axis_name="c", subcore_axis_name="s")


CG = 128
NFULL = PER_W // CG
CTAIL = PER_W - NFULL * CG


def _gather_body(tab_hbm, dst_hbm, src_hbm, xd_hbm, xs_hbm,
                 idxd_v, idxs_v, rda, rdb, rsa, rsb,
                 gda, gdb, gsa, gsb):
    c = lax.axis_index("c")
    s = lax.axis_index("s")
    base = (c * NS + s) * PER_W

    pltpu.sync_copy(dst_hbm.at[pl.ds(base, PER_W)], idxd_v)
    pltpu.sync_copy(src_hbm.at[pl.ds(base, PER_W)], idxs_v)

    def fire(j, rd, rs, gd, gs):
        off = pl.multiple_of(j * CG, 8)
        cpd = pltpu.make_async_copy(tab_hbm.at[idxd_v.at[pl.ds(off, CG)]],
                                    rd, gd)
        cps = pltpu.make_async_copy(tab_hbm.at[idxs_v.at[pl.ds(off, CG)]],
                                    rs, gs)
        cpd.start()
        cps.start()
        return cpd, cps

    def drain(j, rd, rs, gd, gs):
        off = pl.multiple_of(base + j * CG, 8)
        pltpu.make_async_copy(tab_hbm.at[idxd_v.at[pl.ds(0, CG)]], rd, gd).wait()
        pltpu.sync_copy(rd, xd_hbm.at[pl.ds(off, CG)])
        pltpu.make_async_copy(tab_hbm.at[idxs_v.at[pl.ds(0, CG)]], rs, gs).wait()
        pltpu.sync_copy(rs, xs_hbm.at[pl.ds(off, CG)])

    fire(0, rda, rsa, gda, gsa)

    def body(k, carry):
        fire(2 * k + 1, rdb, rsb, gdb, gsb)
        drain(2 * k, rda, rsa, gda, gsa)
        fire(2 * k + 2, rda, rsa, gda, gsa)
        drain(2 * k + 1, rdb, rsb, gdb, gsb)
        return carry

    lax.fori_loop(0, (NFULL - 1) // 2, body, 0)
    drain(NFULL - 1, rda, rsa, gda, gsa)

    toff = pl.multiple_of(NFULL * CG, 8)
    cpd = pltpu.make_async_copy(
        tab_hbm.at[idxd_v.at[pl.ds(toff, CTAIL)]], rdb.at[pl.ds(0, CTAIL)], gdb)
    cps = pltpu.make_async_copy(
        tab_hbm.at[idxs_v.at[pl.ds(toff, CTAIL)]], rsb.at[pl.ds(0, CTAIL)], gsb)
    cpd.start()
    cps.start()
    cpd.wait()
    cps.wait()
    pltpu.sync_copy(rdb.at[pl.ds(0, CTAIL)],
                    xd_hbm.at[pl.ds(base + toff, CTAIL)])
    pltpu.sync_copy(rsb.at[pl.ds(0, CTAIL)],
                    xs_hbm.at[pl.ds(base + toff, CTAIL)])


_gather2 = functools.partial(
    pl.kernel,
    out_type=[jax.ShapeDtypeStruct((E, F), jnp.float32),
              jax.ShapeDtypeStruct((E, F), jnp.float32)],
    mesh=_SC_MESH,
    scratch_types=[pltpu.VMEM((PER_W,), jnp.int32),
                   pltpu.VMEM((PER_W,), jnp.int32),
                   pltpu.VMEM((CG, F), jnp.float32),
                   pltpu.VMEM((CG, F), jnp.float32),
                   pltpu.VMEM((CG, F), jnp.float32),
                   pltpu.VMEM((CG, F), jnp.float32),
                   pltpu.SemaphoreType.DMA,
                   pltpu.SemaphoreType.DMA,
                   pltpu.SemaphoreType.DMA,
                   pltpu.SemaphoreType.DMA],
)(_gather_body)


NCH = PER_W // C


def _scatter_body(msg_hbm, x_hbm, zeros_hbm, dst3d_hbm, p0_hbm, p1_hbm,
                  idx_v, ra, rb, acc_sh, la, lb):
    c = lax.axis_index("c")
    s = lax.axis_index("s")

    roff = s * RT

    @pl.when(c == 0)
    def _():
        pltpu.sync_copy(x_hbm.at[pl.ds(roff, RT)], acc_sh.at[pl.ds(roff, RT)])

    @pl.when(c != 0)
    def _():
        pltpu.sync_copy(zeros_hbm.at[pl.ds(roff, RT)],
                        acc_sh.at[pl.ds(roff, RT)])

    wid = c * NS + s
    pltpu.sync_copy(dst3d_hbm.at[wid], idx_v)

    plsc.subcore_barrier()

    base = wid * PER_W

    def fire(t, rbuf, sem):
        off = pl.multiple_of(base + t * C, 8)
        pltpu.make_async_copy(msg_hbm.at[pl.ds(off, C)], rbuf, sem).start()

    def drain(t, rbuf, sem):
        pltpu.make_async_copy(msg_hbm.at[pl.ds(base, C)], rbuf, sem).wait()
        pltpu.sync_copy(rbuf, acc_sh.at[idx_v.at[t]], add=True)

    fire(0, ra, la)

    def body(k, carry):
        fire(2 * k + 1, rb, lb)
        drain(2 * k, ra, la)
        fire(2 * k + 2, ra, la)
        drain(2 * k + 1, rb, lb)
        return carry

    lax.fori_loop(0, (NCH - 1) // 2, body, 0)
    drain(NCH - 1, ra, la)

    plsc.subcore_barrier()

    @pl.when(c == 0)
    def _():
        pltpu.sync_copy(acc_sh.at[pl.ds(roff, RT)], p0_hbm.at[pl.ds(roff, RT)])

    @pl.when(c != 0)
    def _():
        pltpu.sync_copy(acc_sh.at[pl.ds(roff, RT)], p1_hbm.at[pl.ds(roff, RT)])


_scatter = functools.partial(
    pl.kernel,
    out_type=[jax.ShapeDtypeStruct((N_PAD, F), jnp.float32),
              jax.ShapeDtypeStruct((N_PAD, F), jnp.float32)],
    mesh=_SC_MESH,
    scratch_types=[pltpu.VMEM((NCH, C), jnp.int32),
                   pltpu.VMEM((C, F), jnp.float32),
                   pltpu.VMEM((C, F), jnp.float32),
                   pltpu.VMEM_SHARED((N_PAD, F), jnp.float32),
                   pltpu.SemaphoreType.DMA,
                   pltpu.SemaphoreType.DMA],
)(_scatter_body)


BE1 = 800
BE2 = 1000
BN = 1280


def _p1_body(e_ref, xd_ref, xs_ref, Wpre_ref, bpre_ref,
             Ws1_ref, bs1_ref, Wf1_ref, bf1_ref, We1_ref, be1_ref,
             Ws2e_ref, Wf2e_ref,
             msg_ref, c_ref, d_ref):
    e0 = jnp.tanh(
        jnp.dot(e_ref[...], Wpre_ref[...], preferred_element_type=jnp.float32)
        + bpre_ref[...])
    xd = xd_ref[...]
    xs = xs_ref[...]

    def mm3(W):
        return (jnp.dot(xd, W[:F], preferred_element_type=jnp.float32)
                + jnp.dot(xs, W[F:2 * F], preferred_element_type=jnp.float32)
                + jnp.dot(e0, W[2 * F:], preferred_element_type=jnp.float32))

    s1 = mm3(Ws1_ref[...]) + bs1_ref[...]
    f1 = mm3(Wf1_ref[...]) + bf1_ref[...]
    msg_ref[...] = jax.nn.relu(s1) * jax.nn.sigmoid(f1)
    g1 = jax.nn.sigmoid(mm3(We1_ref[...]) + be1_ref[...])
    e1 = e0 * (1.0 + g1)
    c_ref[...] = jnp.dot(e1, Ws2e_ref[...], preferred_element_type=jnp.float32)
    d_ref[...] = jnp.dot(e1, Wf2e_ref[...], preferred_element_type=jnp.float32)


def _p1(e, xd, xs, W_pre, b_pre, Ws1, bs1, Wf1, bf1, We1, be1, Ws2e, Wf2e):
    full = lambda shp: pl.BlockSpec(shp, lambda i: (0, 0))
    eb = lambda w: pl.BlockSpec((BE1, w), lambda i: (i, 0))
    return pl.pallas_call(
        _p1_body,
        grid=(E // BE1,),
        in_specs=[eb(DE), eb(F), eb(F),
                  full((DE, DPE)), full((1, DPE)),
                  full((2 * F + DPE, F)), full((1, F)),
                  full((2 * F + DPE, F)), full((1, F)),
                  full((2 * F + DPE, DPE)), full((1, DPE)),
                  full((DPE, F)), full((DPE, F))],
        out_specs=[eb(F), eb(F), eb(F)],
        out_shape=[jax.ShapeDtypeStruct((E, F), jnp.float32)] * 3,
    )(e, xd, xs, W_pre, b_pre, Ws1, bs1, Wf1, bf1, We1, be1, Ws2e, Wf2e)


def _add_body(p0_ref, p1_ref, x1_ref, sum_ref):
    i = pl.program_id(0)
    v = p0_ref[...] + p1_ref[...]
    x1_ref[...] = v

    @pl.when(i == 0)
    def _():
        sum_ref[...] = jnp.zeros_like(sum_ref)

    sum_ref[...] += jnp.sum(v, axis=0, keepdims=True)


def _addsum(p0, p1):
    return pl.pallas_call(
        _add_body,
        grid=(N_PAD // BN,),
        in_specs=[pl.BlockSpec((BN, F), lambda i: (i, 0)),
                  pl.BlockSpec((BN, F), lambda i: (i, 0))],
        out_specs=[pl.BlockSpec((BN, F), lambda i: (i, 0)),
                   pl.BlockSpec((1, F), lambda i: (0, 0))],
        out_shape=[jax.ShapeDtypeStruct((N_PAD, F), jnp.float32),
                   jax.ShapeDtypeStruct((1, F), jnp.float32)],
    )(p0, p1)


def _p2_body(x1d_ref, x1s_ref, c_ref, d_ref,
             Ws2n_ref, bs2_ref, Wf2n_ref, bf2_ref,
             sumx1_ref, Wout_ref, bout_ref, out_ref, acc_ref):
    i = pl.program_id(0)

    @pl.when(i == 0)
    def _():
        acc_ref[...] = jnp.zeros_like(acc_ref)

    x1d = x1d_ref[...]
    x1s = x1s_ref[...]
    Ws2n = Ws2n_ref[...]
    Wf2n = Wf2n_ref[...]
    s2 = (jnp.dot(x1d, Ws2n[:F], preferred_element_type=jnp.float32)
          + jnp.dot(x1s, Ws2n[F:], preferred_element_type=jnp.float32)
          + c_ref[...] + bs2_ref[...])
    f2 = (jnp.dot(x1d, Wf2n[:F], preferred_element_type=jnp.float32)
          + jnp.dot(x1s, Wf2n[F:], preferred_element_type=jnp.float32)
          + d_ref[...] + bf2_ref[...])
    m = jax.nn.relu(s2) * jax.nn.sigmoid(f2)
    acc_ref[...] += jnp.sum(m, axis=0, keepdims=True)

    @pl.when(i == pl.num_programs(0) - 1)
    def _():
        pooled = acc_ref[...] + sumx1_ref[...]
        logits = (jnp.dot(pooled, Wout_ref[...],
                          preferred_element_type=jnp.float32) + bout_ref[...])
        mx = jnp.max(logits, axis=-1, keepdims=True)
        ex = jnp.exp(logits - mx)
        out_ref[...] = ex / jnp.sum(ex, axis=-1, keepdims=True)


def _p2(x1d, x1s, cc, dd, Ws2n, bs2, Wf2n, bf2, sumx1, W_out, b_out):
    full = lambda shp: pl.BlockSpec(shp, lambda i: (0, 0))
    eb = lambda w: pl.BlockSpec((BE2, w), lambda i: (i, 0))
    return pl.pallas_call(
        _p2_body,
        grid=(E // BE2,),
        in_specs=[eb(F), eb(F), eb(F), eb(F),
                  full((2 * F, F)), full((1, F)),
                  full((2 * F, F)), full((1, F)),
                  full((1, F)), full((F, 32)), full((1, 32))],
        out_specs=full((1, 32)),
        out_shape=jax.ShapeDtypeStruct((1, 32), jnp.float32),
        scratch_shapes=[pltpu.VMEM((1, F), jnp.float32)],
    )(x1d, x1s, cc, dd, Ws2n, bs2, Wf2n, bf2, sumx1, W_out, b_out)


def kernel(x, edge_index, e,
           W_pre, b_pre,
           Wf1, bf1, Ws1, bs1, We1, be1,
           Wf2, bf2, Ws2, bs2, We2, be2,
           W_out, b_out):
    dst = edge_index[0]
    src = edge_index[1]
    zeros = jnp.zeros((N_PAD, F), jnp.float32)
    x_pad = jnp.pad(x, ((0, N_PAD - N), (0, 0)))

    xd, xs = _gather2(x, dst, src)
    msg1, cc, dd = _p1(e, xd, xs,
                       W_pre, b_pre.reshape(1, DPE),
                       Ws1, bs1.reshape(1, F),
                       Wf1, bf1.reshape(1, F),
                       We1, be1.reshape(1, DPE),
                       Ws2[2 * F:], Wf2[2 * F:])
    p0, p1 = _scatter(msg1, x_pad, zeros, dst.reshape(NW, NCH, C))
    x1, sumx1 = _addsum(p0, p1)
    x1d, x1s = _gather2(x1, dst, src)
    out = _p2(x1d, x1s, cc, dd,
              Ws2[:2 * F], bs2.reshape(1, F),
              Wf2[:2 * F], bf2.reshape(1, F),
              sumx1, W_out, b_out.reshape(1, 32))
    return out.reshape(32)


# speedup vs baseline: 3.1780x; 1.3636x over previous
import functools

import jax
import jax.numpy as jnp
from jax import lax
from jax.experimental import pallas as pl
from jax.experimental.pallas import tpu as pltpu
from jax.experimental.pallas import tpu_sc as plsc

N = 10000
E = 160000
F = 128
DE = 16
DPE = 768

NC = 2
NS = 16
NW = NC * NS
PER_W = E // NW
C = 40
G_ITERS = PER_W // C
N_PAD = 10240
RT = N_PAD // NS

_SC_MESH = plsc.VectorSubcoreMesh(core_axis_name="c", subcore_axis_name="s")


CG = 128
NFULL = PER_W // CG
CTAIL = PER_W - NFULL * CG


def _gather_body(tab_hbm, dst_hbm, src_hbm, xd_hbm, xs_hbm,
                 idxd_v, idxs_v, rda, rdb, rsa, rsb,
                 gda, gdb, gsa, gsb):
    c = lax.axis_index("c")
    s = lax.axis_index("s")
    base = (c * NS + s) * PER_W

    pltpu.sync_copy(dst_hbm.at[pl.ds(base, PER_W)], idxd_v)
    pltpu.sync_copy(src_hbm.at[pl.ds(base, PER_W)], idxs_v)

    def fire(j, rd, rs, gd, gs):
        off = pl.multiple_of(j * CG, 8)
        cpd = pltpu.make_async_copy(tab_hbm.at[idxd_v.at[pl.ds(off, CG)]],
                                    rd, gd)
        cps = pltpu.make_async_copy(tab_hbm.at[idxs_v.at[pl.ds(off, CG)]],
                                    rs, gs)
        cpd.start()
        cps.start()
        return cpd, cps

    def drain(j, rd, rs, gd, gs):
        off = pl.multiple_of(base + j * CG, 8)
        pltpu.make_async_copy(tab_hbm.at[idxd_v.at[pl.ds(0, CG)]], rd, gd).wait()
        pltpu.sync_copy(rd, xd_hbm.at[pl.ds(off, CG)])
        pltpu.make_async_copy(tab_hbm.at[idxs_v.at[pl.ds(0, CG)]], rs, gs).wait()
        pltpu.sync_copy(rs, xs_hbm.at[pl.ds(off, CG)])

    fire(0, rda, rsa, gda, gsa)

    def body(k, carry):
        fire(2 * k + 1, rdb, rsb, gdb, gsb)
        drain(2 * k, rda, rsa, gda, gsa)
        fire(2 * k + 2, rda, rsa, gda, gsa)
        drain(2 * k + 1, rdb, rsb, gdb, gsb)
        return carry

    lax.fori_loop(0, (NFULL - 1) // 2, body, 0)
    drain(NFULL - 1, rda, rsa, gda, gsa)

    toff = pl.multiple_of(NFULL * CG, 8)
    cpd = pltpu.make_async_copy(
        tab_hbm.at[idxd_v.at[pl.ds(toff, CTAIL)]], rdb.at[pl.ds(0, CTAIL)], gdb)
    cps = pltpu.make_async_copy(
        tab_hbm.at[idxs_v.at[pl.ds(toff, CTAIL)]], rsb.at[pl.ds(0, CTAIL)], gsb)
    cpd.start()
    cps.start()
    cpd.wait()
    cps.wait()
    pltpu.sync_copy(rdb.at[pl.ds(0, CTAIL)],
                    xd_hbm.at[pl.ds(base + toff, CTAIL)])
    pltpu.sync_copy(rsb.at[pl.ds(0, CTAIL)],
                    xs_hbm.at[pl.ds(base + toff, CTAIL)])


_gather2 = functools.partial(
    pl.kernel,
    out_type=[jax.ShapeDtypeStruct((E, F), jnp.float32),
              jax.ShapeDtypeStruct((E, F), jnp.float32)],
    mesh=_SC_MESH,
    scratch_types=[pltpu.VMEM((PER_W,), jnp.int32),
                   pltpu.VMEM((PER_W,), jnp.int32),
                   pltpu.VMEM((CG, F), jnp.float32),
                   pltpu.VMEM((CG, F), jnp.float32),
                   pltpu.VMEM((CG, F), jnp.float32),
                   pltpu.VMEM((CG, F), jnp.float32),
                   pltpu.SemaphoreType.DMA,
                   pltpu.SemaphoreType.DMA,
                   pltpu.SemaphoreType.DMA,
                   pltpu.SemaphoreType.DMA],
)(_gather_body)


NCH = PER_W // C


def _scatter_body(msg_hbm, x_hbm, zeros_hbm, dst3d_hbm, p0_hbm, p1_hbm,
                  idx_v, ra, rb, acc_sh, la, lb):
    c = lax.axis_index("c")
    s = lax.axis_index("s")

    roff = s * RT

    @pl.when(c == 0)
    def _():
        pltpu.sync_copy(x_hbm.at[pl.ds(roff, RT)], acc_sh.at[pl.ds(roff, RT)])

    @pl.when(c != 0)
    def _():
        pltpu.sync_copy(zeros_hbm.at[pl.ds(roff, RT)],
                        acc_sh.at[pl.ds(roff, RT)])

    wid = c * NS + s
    pltpu.sync_copy(dst3d_hbm.at[wid], idx_v)

    plsc.subcore_barrier()

    base = wid * PER_W

    def fire(t, rbuf, sem):
        off = pl.multiple_of(base + t * C, 8)
        pltpu.make_async_copy(msg_hbm.at[pl.ds(off, C)], rbuf, sem).start()

    def drain(t, rbuf, sem):
        pltpu.make_async_copy(msg_hbm.at[pl.ds(base, C)], rbuf, sem).wait()
        pltpu.sync_copy(rbuf, acc_sh.at[idx_v.at[t]], add=True)

    fire(0, ra, la)

    def body(k, carry):
        fire(2 * k + 1, rb, lb)
        drain(2 * k, ra, la)
        fire(2 * k + 2, ra, la)
        drain(2 * k + 1, rb, lb)
        return carry

    lax.fori_loop(0, (NCH - 1) // 2, body, 0)
    drain(NCH - 1, ra, la)

    plsc.subcore_barrier()

    @pl.when(c == 0)
    def _():
        pltpu.sync_copy(acc_sh.at[pl.ds(roff, RT)], p0_hbm.at[pl.ds(roff, RT)])

    @pl.when(c != 0)
    def _():
        pltpu.sync_copy(acc_sh.at[pl.ds(roff, RT)], p1_hbm.at[pl.ds(roff, RT)])


_scatter = functools.partial(
    pl.kernel,
    out_type=[jax.ShapeDtypeStruct((N_PAD, F), jnp.float32),
              jax.ShapeDtypeStruct((N_PAD, F), jnp.float32)],
    mesh=_SC_MESH,
    scratch_types=[pltpu.VMEM((NCH, C), jnp.int32),
                   pltpu.VMEM((C, F), jnp.float32),
                   pltpu.VMEM((C, F), jnp.float32),
                   pltpu.VMEM_SHARED((N_PAD, F), jnp.float32),
                   pltpu.SemaphoreType.DMA,
                   pltpu.SemaphoreType.DMA],
)(_scatter_body)


BE1 = 1600
BE2 = 1000
BN = 1280
Z = 2 * F + DPE


def _sig(v):
    return 0.5 + 0.5 * jnp.tanh(0.5 * v)


def _p1_body(e_ref, xd_ref, xs_ref, Wpre_ref, bpre_ref,
             Wcat_ref, bcat_ref, Wcd_ref,
             msg_ref, c_ref, d_ref):
    bf = jnp.bfloat16
    e0 = jnp.tanh(
        jnp.dot(e_ref[...].astype(bf), Wpre_ref[...],
                preferred_element_type=jnp.float32) + bpre_ref[...])
    z = jnp.concatenate(
        [xd_ref[...].astype(bf), xs_ref[...].astype(bf), e0.astype(bf)],
        axis=1)
    u = (jnp.dot(z, Wcat_ref[...], preferred_element_type=jnp.float32)
         + bcat_ref[...])
    msg_ref[...] = jax.nn.relu(u[:, :F]) * _sig(u[:, F:2 * F])
    e1 = (e0 * (1.0 + _sig(u[:, 2 * F:]))).astype(bf)
    cd = jnp.dot(e1, Wcd_ref[...], preferred_element_type=jnp.float32)
    c_ref[...] = cd[:, :F]
    d_ref[...] = cd[:, F:]


def _p1(e, xd, xs, W_pre, b_pre, Wcat, bcat, Wcd):
    full = lambda shp: pl.BlockSpec(shp, lambda i: (0, 0))
    eb = lambda w: pl.BlockSpec((BE1, w), lambda i: (i, 0))
    return pl.pallas_call(
        _p1_body,
        grid=(E // BE1,),
        in_specs=[eb(DE), eb(F), eb(F),
                  full((DE, DPE)), full((1, DPE)),
                  full((Z, Z)), full((1, Z)),
                  full((DPE, 2 * F))],
        out_specs=[eb(F), eb(F), eb(F)],
        out_shape=[jax.ShapeDtypeStruct((E, F), jnp.float32)] * 3,
    )(e, xd, xs, W_pre, b_pre, Wcat, bcat, Wcd)


def _add_body(p0_ref, p1_ref, x1_ref, sum_ref):
    i = pl.program_id(0)
    v = p0_ref[...] + p1_ref[...]
    x1_ref[...] = v

    @pl.when(i == 0)
    def _():
        sum_ref[...] = jnp.zeros_like(sum_ref)

    sum_ref[...] += jnp.sum(v, axis=0, keepdims=True)


def _addsum(p0, p1):
    return pl.pallas_call(
        _add_body,
        grid=(N_PAD // BN,),
        in_specs=[pl.BlockSpec((BN, F), lambda i: (i, 0)),
                  pl.BlockSpec((BN, F), lambda i: (i, 0))],
        out_specs=[pl.BlockSpec((BN, F), lambda i: (i, 0)),
                   pl.BlockSpec((1, F), lambda i: (0, 0))],
        out_shape=[jax.ShapeDtypeStruct((N_PAD, F), jnp.float32),
                   jax.ShapeDtypeStruct((1, F), jnp.float32)],
    )(p0, p1)


def _p2_body(x1d_ref, x1s_ref, c_ref, d_ref,
             Wcat2_ref, bcat2_ref,
             sumx1_ref, Wout_ref, bout_ref, out_ref, acc_ref):
    i = pl.program_id(0)

    @pl.when(i == 0)
    def _():
        acc_ref[...] = jnp.zeros_like(acc_ref)

    z2 = jnp.concatenate([x1d_ref[...].astype(jnp.bfloat16),
                          x1s_ref[...].astype(jnp.bfloat16)], axis=1)
    u = (jnp.dot(z2, Wcat2_ref[...], preferred_element_type=jnp.float32)
         + bcat2_ref[...])
    s2 = u[:, :F] + c_ref[...]
    f2 = u[:, F:] + d_ref[...]
    m = jax.nn.relu(s2) * _sig(f2)
    acc_ref[...] += jnp.sum(m, axis=0, keepdims=True)

    @pl.when(i == pl.num_programs(0) - 1)
    def _():
        pooled = acc_ref[...] + sumx1_ref[...]
        logits = (jnp.dot(pooled, Wout_ref[...],
                          preferred_element_type=jnp.float32) + bout_ref[...])
        mx = jnp.max(logits, axis=-1, keepdims=True)
        ex = jnp.exp(logits - mx)
        out_ref[...] = ex / jnp.sum(ex, axis=-1, keepdims=True)


def _p2(x1d, x1s, cc, dd, Wcat2, bcat2, sumx1, W_out, b_out):
    full = lambda shp: pl.BlockSpec(shp, lambda i: (0, 0))
    eb = lambda w: pl.BlockSpec((BE2, w), lambda i: (i, 0))
    return pl.pallas_call(
        _p2_body,
        grid=(E // BE2,),
        in_specs=[eb(F), eb(F), eb(F), eb(F),
                  full((2 * F, 2 * F)), full((1, 2 * F)),
                  full((1, F)), full((F, 32)), full((1, 32))],
        out_specs=full((1, 32)),
        out_shape=jax.ShapeDtypeStruct((1, 32), jnp.float32),
        scratch_shapes=[pltpu.VMEM((1, F), jnp.float32)],
    )(x1d, x1s, cc, dd, Wcat2, bcat2, sumx1, W_out, b_out)


def kernel(x, edge_index, e,
           W_pre, b_pre,
           Wf1, bf1, Ws1, bs1, We1, be1,
           Wf2, bf2, Ws2, bs2, We2, be2,
           W_out, b_out):
    dst = edge_index[0]
    src = edge_index[1]
    zeros = jnp.zeros((N_PAD, F), jnp.float32)
    x_pad = jnp.pad(x, ((0, N_PAD - N), (0, 0)))

    bf = jnp.bfloat16
    Wcat = jnp.concatenate([Ws1, Wf1, We1], axis=1).astype(bf)
    bcat = jnp.concatenate([bs1, bf1, be1]).reshape(1, Z)
    Wcd = jnp.concatenate([Ws2[2 * F:], Wf2[2 * F:]], axis=1).astype(bf)
    Wcat2 = jnp.concatenate([Ws2[:2 * F], Wf2[:2 * F]], axis=1).astype(bf)
    bcat2 = jnp.concatenate([bs2, bf2]).reshape(1, 2 * F)

    xd, xs = _gather2(x, dst, src)
    msg1, cc, dd = _p1(e, xd, xs, W_pre.astype(bf), b_pre.reshape(1, DPE),
                       Wcat, bcat, Wcd)
    p0, p1 = _scatter(msg1, x_pad, zeros, dst.reshape(NW, NCH, C))
    x1, sumx1 = _addsum(p0, p1)
    x1d, x1s = _gather2(x1, dst, src)
    out = _p2(x1d, x1s, cc, dd, Wcat2, bcat2,
              sumx1, W_out, b_out.reshape(1, 32))
    return out.reshape(32)


# speedup vs baseline: 3.2375x; 1.0187x over previous
import functools

import jax
import jax.numpy as jnp
from jax import lax
from jax.experimental import pallas as pl
from jax.experimental.pallas import tpu as pltpu
from jax.experimental.pallas import tpu_sc as plsc

N = 10000
E = 160000
F = 128
DE = 16
DPE = 768

NC = 2
NS = 16
NW = NC * NS
PER_W = E // NW
C = 40
G_ITERS = PER_W // C
N_PAD = 10240
RT = N_PAD // NS

_SC_MESH = plsc.VectorSubcoreMesh(core_axis_name="c", subcore_axis_name="s")


CG = 128
EH = E // 2
CH_H = EH // CG
T_FULL = CH_H // NW
W_EXTRA = CH_H - T_FULL * NW


def _gather_body(tab_hbm, dst_hbm, src_hbm, xd_hbm, xs_hbm,
                 idxa, idxb, rda, rdb, rsa, rsb,
                 gda, gdb, gsa, gsb):
    c = lax.axis_index("c")
    s = lax.axis_index("s")
    w = c * NS + s

    def fire(t, idx, rd, rs, gd, gs):
        off = pl.multiple_of((w + t * NW) * CG, 8)
        pltpu.sync_copy(dst_hbm.at[pl.ds(off, CG)], idx.at[0])
        pltpu.sync_copy(src_hbm.at[pl.ds(off, CG)], idx.at[1])
        pltpu.make_async_copy(tab_hbm.at[idx.at[0]], rd, gd).start()
        pltpu.make_async_copy(tab_hbm.at[idx.at[1]], rs, gs).start()

    def drain(t, idx, rd, rs, gd, gs):
        off = pl.multiple_of((w + t * NW) * CG, 8)
        pltpu.make_async_copy(tab_hbm.at[idx.at[0]], rd, gd).wait()
        pltpu.sync_copy(rd, xd_hbm.at[pl.ds(off, CG)])
        pltpu.make_async_copy(tab_hbm.at[idx.at[1]], rs, gs).wait()
        pltpu.sync_copy(rs, xs_hbm.at[pl.ds(off, CG)])

    fire(0, idxa, rda, rsa, gda, gsa)

    def body(k, carry):
        fire(2 * k + 1, idxb, rdb, rsb, gdb, gsb)
        drain(2 * k, idxa, rda, rsa, gda, gsa)
        fire(2 * k + 2, idxa, rda, rsa, gda, gsa)
        drain(2 * k + 1, idxb, rdb, rsb, gdb, gsb)
        return carry

    lax.fori_loop(0, (T_FULL - 1) // 2, body, 0)
    drain(T_FULL - 1, idxa, rda, rsa, gda, gsa)

    @pl.when(w < W_EXTRA)
    def _():
        fire(T_FULL, idxb, rdb, rsb, gdb, gsb)
        drain(T_FULL, idxb, rdb, rsb, gdb, gsb)


_gather2 = functools.partial(
    pl.kernel,
    out_type=[jax.ShapeDtypeStruct((EH, F), jnp.float32),
              jax.ShapeDtypeStruct((EH, F), jnp.float32)],
    mesh=_SC_MESH,
    scratch_types=[pltpu.VMEM((2, CG), jnp.int32),
                   pltpu.VMEM((2, CG), jnp.int32),
                   pltpu.VMEM((CG, F), jnp.float32),
                   pltpu.VMEM((CG, F), jnp.float32),
                   pltpu.VMEM((CG, F), jnp.float32),
                   pltpu.VMEM((CG, F), jnp.float32),
                   pltpu.SemaphoreType.DMA,
                   pltpu.SemaphoreType.DMA,
                   pltpu.SemaphoreType.DMA,
                   pltpu.SemaphoreType.DMA],
)(_gather_body)


NCH = PER_W // C


def _scatter_body(msga_hbm, msgb_hbm, x_hbm, zeros_hbm, dst3d_hbm,
                  p0_hbm, p1_hbm,
                  idx_v, ra, rb, acc_sh, la, lb):
    c = lax.axis_index("c")
    s = lax.axis_index("s")

    roff = s * RT

    @pl.when(c == 0)
    def _():
        pltpu.sync_copy(x_hbm.at[pl.ds(roff, RT)], acc_sh.at[pl.ds(roff, RT)])

    @pl.when(c != 0)
    def _():
        pltpu.sync_copy(zeros_hbm.at[pl.ds(roff, RT)],
                        acc_sh.at[pl.ds(roff, RT)])

    wid = c * NS + s
    pltpu.sync_copy(dst3d_hbm.at[wid], idx_v)

    plsc.subcore_barrier()

    base = (wid % NS) * PER_W

    def fire(t, rbuf, sem):
        off = pl.multiple_of(base + t * C, 8)

        @pl.when(c == 0)
        def _():
            pltpu.make_async_copy(msga_hbm.at[pl.ds(off, C)], rbuf, sem).start()

        @pl.when(c != 0)
        def _():
            pltpu.make_async_copy(msgb_hbm.at[pl.ds(off, C)], rbuf, sem).start()

    def drain(t, rbuf, sem):
        pltpu.make_async_copy(msga_hbm.at[pl.ds(base, C)], rbuf, sem).wait()
        pltpu.sync_copy(rbuf, acc_sh.at[idx_v.at[t]], add=True)

    fire(0, ra, la)

    def body(k, carry):
        fire(2 * k + 1, rb, lb)
        drain(2 * k, ra, la)
        fire(2 * k + 2, ra, la)
        drain(2 * k + 1, rb, lb)
        return carry

    lax.fori_loop(0, (NCH - 1) // 2, body, 0)
    drain(NCH - 1, ra, la)

    plsc.subcore_barrier()

    @pl.when(c == 0)
    def _():
        pltpu.sync_copy(acc_sh.at[pl.ds(roff, RT)], p0_hbm.at[pl.ds(roff, RT)])

    @pl.when(c != 0)
    def _():
        pltpu.sync_copy(acc_sh.at[pl.ds(roff, RT)], p1_hbm.at[pl.ds(roff, RT)])


_scatter = functools.partial(
    pl.kernel,
    out_type=[jax.ShapeDtypeStruct((N_PAD, F), jnp.float32),
              jax.ShapeDtypeStruct((N_PAD, F), jnp.float32)],
    mesh=_SC_MESH,
    scratch_types=[pltpu.VMEM((NCH, C), jnp.int32),
                   pltpu.VMEM((C, F), jnp.float32),
                   pltpu.VMEM((C, F), jnp.float32),
                   pltpu.VMEM_SHARED((N_PAD, F), jnp.float32),
                   pltpu.SemaphoreType.DMA,
                   pltpu.SemaphoreType.DMA],
)(_scatter_body)


BE1 = 1600
BE2 = 1000
BN = 1280
Z = 2 * F + DPE


def _sig(v):
    return 0.5 + 0.5 * jnp.tanh(0.5 * v)


def _p1_body(e_ref, xd_ref, xs_ref, Wpre_ref, bpre_ref,
             Wcat_ref, bcat_ref, Wcd_ref,
             msg_ref, c_ref, d_ref):
    bf = jnp.bfloat16
    e0 = jnp.tanh(
        jnp.dot(e_ref[...].astype(bf), Wpre_ref[...],
                preferred_element_type=jnp.float32) + bpre_ref[...])
    z = jnp.concatenate([xd_ref[...].astype(bf), xs_ref[...].astype(bf),
                         e0.astype(bf)], axis=1)
    u = (jnp.dot(z, Wcat_ref[...], preferred_element_type=jnp.float32)
         + bcat_ref[...])
    msg_ref[...] = jax.nn.relu(u[:, :F]) * _sig(u[:, F:2 * F])
    e1 = (e0 * (1.0 + _sig(u[:, 2 * F:]))).astype(bf)
    cd = jnp.dot(e1, Wcd_ref[...], preferred_element_type=jnp.float32)
    c_ref[...] = cd[:, :F]
    d_ref[...] = cd[:, F:]


def _p1(e, xd, xs, W_pre, b_pre, Wcat, bcat, Wcd):
    full = lambda shp: pl.BlockSpec(shp, lambda i: (0, 0))
    eb = lambda w: pl.BlockSpec((BE1, w), lambda i: (i, 0))
    return pl.pallas_call(
        _p1_body,
        grid=(EH // BE1,),
        in_specs=[eb(DE), eb(F), eb(F),
                  full((DE, DPE)), full((1, DPE)),
                  full((Z, Z)), full((1, Z)),
                  full((DPE, 2 * F))],
        out_specs=[eb(F), eb(F), eb(F)],
        out_shape=[jax.ShapeDtypeStruct((EH, F), jnp.float32)] * 3,
    )(e, xd, xs, W_pre, b_pre, Wcat, bcat, Wcd)


def _add_body(p0_ref, p1_ref, x1_ref, sum_ref):
    i = pl.program_id(0)
    v = p0_ref[...] + p1_ref[...]
    x1_ref[...] = v

    @pl.when(i == 0)
    def _():
        sum_ref[...] = jnp.zeros_like(sum_ref)

    sum_ref[...] += jnp.sum(v, axis=0, keepdims=True)


def _addsum(p0, p1):
    return pl.pallas_call(
        _add_body,
        grid=(N_PAD // BN,),
        in_specs=[pl.BlockSpec((BN, F), lambda i: (i, 0)),
                  pl.BlockSpec((BN, F), lambda i: (i, 0))],
        out_specs=[pl.BlockSpec((BN, F), lambda i: (i, 0)),
                   pl.BlockSpec((1, F), lambda i: (0, 0))],
        out_shape=[jax.ShapeDtypeStruct((N_PAD, F), jnp.float32),
                   jax.ShapeDtypeStruct((1, F), jnp.float32)],
    )(p0, p1)


def _p2_body(x1d_ref, x1s_ref, c_ref, d_ref,
             Wcat2_ref, bcat2_ref, acc_ref):
    i = pl.program_id(0)

    @pl.when(i == 0)
    def _():
        acc_ref[...] = jnp.zeros_like(acc_ref)

    z2 = jnp.concatenate([x1d_ref[...].astype(jnp.bfloat16),
                          x1s_ref[...].astype(jnp.bfloat16)], axis=1)
    u = (jnp.dot(z2, Wcat2_ref[...], preferred_element_type=jnp.float32)
         + bcat2_ref[...])
    s2 = u[:, :F] + c_ref[...]
    f2 = u[:, F:] + d_ref[...]
    m = jax.nn.relu(s2) * _sig(f2)
    acc_ref[...] += jnp.sum(m, axis=0, keepdims=True)


def _p2(x1d, x1s, cc, dd, Wcat2, bcat2):
    full = lambda shp: pl.BlockSpec(shp, lambda i: (0, 0))
    eb = lambda w: pl.BlockSpec((BE2, w), lambda i: (i, 0))
    return pl.pallas_call(
        _p2_body,
        grid=(EH // BE2,),
        in_specs=[eb(F), eb(F), eb(F), eb(F),
                  full((2 * F, 2 * F)), full((1, 2 * F))],
        out_specs=full((1, F)),
        out_shape=jax.ShapeDtypeStruct((1, F), jnp.float32),
    )(x1d, x1s, cc, dd, Wcat2, bcat2)


def _finish_body(sumx1_ref, acca_ref, accb_ref, Wout_ref, bout_ref, out_ref):
    pooled = sumx1_ref[...] + acca_ref[...] + accb_ref[...]
    logits = (jnp.dot(pooled, Wout_ref[...],
                      preferred_element_type=jnp.float32) + bout_ref[...])
    mx = jnp.max(logits, axis=-1, keepdims=True)
    ex = jnp.exp(logits - mx)
    out_ref[...] = ex / jnp.sum(ex, axis=-1, keepdims=True)


def _finish(sumx1, acca, accb, W_out, b_out):
    return pl.pallas_call(
        _finish_body,
        out_shape=jax.ShapeDtypeStruct((1, 32), jnp.float32),
    )(sumx1, acca, accb, W_out, b_out)


def kernel(x, edge_index, e,
           W_pre, b_pre,
           Wf1, bf1, Ws1, bs1, We1, be1,
           Wf2, bf2, Ws2, bs2, We2, be2,
           W_out, b_out):
    dst = edge_index[0]
    src = edge_index[1]
    zeros = jnp.zeros((N_PAD, F), jnp.float32)
    x_pad = jnp.pad(x, ((0, N_PAD - N), (0, 0)))

    bf = jnp.bfloat16
    Wcat = jnp.concatenate([Ws1, Wf1, We1], axis=1).astype(bf)
    bcat = jnp.concatenate([bs1, bf1, be1]).reshape(1, Z)
    Wcd = jnp.concatenate([Ws2[2 * F:], Wf2[2 * F:]], axis=1).astype(bf)
    Wcat2 = jnp.concatenate([Ws2[:2 * F], Wf2[:2 * F]], axis=1).astype(bf)
    bcat2 = jnp.concatenate([bs2, bf2]).reshape(1, 2 * F)

    dsts = (dst[:EH], dst[EH:])
    srcs = (src[:EH], src[EH:])
    es = (e[:EH], e[EH:])
    Wpre16 = W_pre.astype(bf)
    bpre = b_pre.reshape(1, DPE)

    g1 = [_gather2(x, dsts[h], srcs[h]) for h in range(2)]
    r1 = [_p1(es[h], g1[h][0], g1[h][1], Wpre16, bpre, Wcat, bcat, Wcd)
          for h in range(2)]
    p0, p1 = _scatter(r1[0][0], r1[1][0], x_pad, zeros,
                      dst.reshape(NW, NCH, C))
    x1, sumx1 = _addsum(p0, p1)
    g2 = [_gather2(x1, dsts[h], srcs[h]) for h in range(2)]
    accs = [_p2(g2[h][0], g2[h][1], r1[h][1], r1[h][2], Wcat2, bcat2)
            for h in range(2)]
    out = _finish(sumx1, accs[0], accs[1], W_out, b_out.reshape(1, 32))
    return out.reshape(32)


# speedup vs baseline: 3.2982x; 1.0188x over previous
import functools

import jax
import jax.numpy as jnp
from jax import lax
from jax.experimental import pallas as pl
from jax.experimental.pallas import tpu as pltpu
from jax.experimental.pallas import tpu_sc as plsc

N = 10000
E = 160000
F = 128
DE = 16
DPE = 768

NC = 2
NS = 16
NW = NC * NS
PER_W = E // NW
C = 40
G_ITERS = PER_W // C
RT = 640
RT_LAST = N - 15 * RT

_SC_MESH = plsc.VectorSubcoreMesh(core_axis_name="c", subcore_axis_name="s")


CG = 128
EH = E // 2
CH_H = EH // CG
T_FULL = CH_H // NW
W_EXTRA = CH_H - T_FULL * NW


def _gather_body(tab_hbm, dst_hbm, src_hbm, xd_hbm, xs_hbm,
                 idxa, idxb, rda, rdb, rsa, rsb,
                 gda, gdb, gsa, gsb):
    c = lax.axis_index("c")
    s = lax.axis_index("s")
    w = c * NS + s

    def fire(t, idx, rd, rs, gd, gs):
        off = pl.multiple_of((w + t * NW) * CG, 8)
        pltpu.sync_copy(dst_hbm.at[pl.ds(off, CG)], idx.at[0])
        pltpu.sync_copy(src_hbm.at[pl.ds(off, CG)], idx.at[1])
        pltpu.make_async_copy(tab_hbm.at[idx.at[0]], rd, gd).start()
        pltpu.make_async_copy(tab_hbm.at[idx.at[1]], rs, gs).start()

    def drain(t, idx, rd, rs, gd, gs):
        off = pl.multiple_of((w + t * NW) * CG, 8)
        pltpu.make_async_copy(tab_hbm.at[idx.at[0]], rd, gd).wait()
        pltpu.sync_copy(rd, xd_hbm.at[pl.ds(off, CG)])
        pltpu.make_async_copy(tab_hbm.at[idx.at[1]], rs, gs).wait()
        pltpu.sync_copy(rs, xs_hbm.at[pl.ds(off, CG)])

    fire(0, idxa, rda, rsa, gda, gsa)

    def body(k, carry):
        fire(2 * k + 1, idxb, rdb, rsb, gdb, gsb)
        drain(2 * k, idxa, rda, rsa, gda, gsa)
        fire(2 * k + 2, idxa, rda, rsa, gda, gsa)
        drain(2 * k + 1, idxb, rdb, rsb, gdb, gsb)
        return carry

    lax.fori_loop(0, (T_FULL - 1) // 2, body, 0)
    drain(T_FULL - 1, idxa, rda, rsa, gda, gsa)

    @pl.when(w < W_EXTRA)
    def _():
        fire(T_FULL, idxb, rdb, rsb, gdb, gsb)
        drain(T_FULL, idxb, rdb, rsb, gdb, gsb)


_gather2 = functools.partial(
    pl.kernel,
    out_type=[jax.ShapeDtypeStruct((EH, F), jnp.float32),
              jax.ShapeDtypeStruct((EH, F), jnp.float32)],
    mesh=_SC_MESH,
    scratch_types=[pltpu.VMEM((2, CG), jnp.int32),
                   pltpu.VMEM((2, CG), jnp.int32),
                   pltpu.VMEM((CG, F), jnp.float32),
                   pltpu.VMEM((CG, F), jnp.float32),
                   pltpu.VMEM((CG, F), jnp.float32),
                   pltpu.VMEM((CG, F), jnp.float32),
                   pltpu.SemaphoreType.DMA,
                   pltpu.SemaphoreType.DMA,
                   pltpu.SemaphoreType.DMA,
                   pltpu.SemaphoreType.DMA],
)(_gather_body)


NCH = PER_W // C


def _scatter_body(msga_hbm, msgb_hbm, x_hbm, dst3d_hbm,
                  p0_hbm, p1_hbm,
                  idx_v, ra, rb, acc_sh, la, lb):
    c = lax.axis_index("c")
    s = lax.axis_index("s")

    roff = s * RT

    @pl.when(s < NS - 1)
    def _():
        pltpu.sync_copy(x_hbm.at[pl.ds(roff, RT)], acc_sh.at[pl.ds(roff, RT)])

    @pl.when(s == NS - 1)
    def _():
        pltpu.sync_copy(x_hbm.at[pl.ds(roff, RT_LAST)],
                        acc_sh.at[pl.ds(roff, RT_LAST)])

    wid = c * NS + s
    pltpu.sync_copy(dst3d_hbm.at[wid], idx_v)

    plsc.subcore_barrier()

    base = (wid % NS) * PER_W

    def fire(t, rbuf, sem):
        off = pl.multiple_of(base + t * C, 8)

        @pl.when(c == 0)
        def _():
            pltpu.make_async_copy(msga_hbm.at[pl.ds(off, C)], rbuf, sem).start()

        @pl.when(c != 0)
        def _():
            pltpu.make_async_copy(msgb_hbm.at[pl.ds(off, C)], rbuf, sem).start()

    def drain(t, rbuf, sem):
        pltpu.make_async_copy(msga_hbm.at[pl.ds(base, C)], rbuf, sem).wait()
        pltpu.sync_copy(rbuf, acc_sh.at[idx_v.at[t]], add=True)

    fire(0, ra, la)

    def body(k, carry):
        fire(2 * k + 1, rb, lb)
        drain(2 * k, ra, la)
        fire(2 * k + 2, ra, la)
        drain(2 * k + 1, rb, lb)
        return carry

    lax.fori_loop(0, (NCH - 1) // 2, body, 0)
    drain(NCH - 1, ra, la)

    plsc.subcore_barrier()

    @pl.when((c == 0) & (s < NS - 1))
    def _():
        pltpu.sync_copy(acc_sh.at[pl.ds(roff, RT)], p0_hbm.at[pl.ds(roff, RT)])

    @pl.when((c == 0) & (s == NS - 1))
    def _():
        pltpu.sync_copy(acc_sh.at[pl.ds(roff, RT_LAST)],
                        p0_hbm.at[pl.ds(roff, RT_LAST)])

    @pl.when((c != 0) & (s < NS - 1))
    def _():
        pltpu.sync_copy(acc_sh.at[pl.ds(roff, RT)], p1_hbm.at[pl.ds(roff, RT)])

    @pl.when((c != 0) & (s == NS - 1))
    def _():
        pltpu.sync_copy(acc_sh.at[pl.ds(roff, RT_LAST)],
                        p1_hbm.at[pl.ds(roff, RT_LAST)])


_scatter = functools.partial(
    pl.kernel,
    out_type=[jax.ShapeDtypeStruct((N, F), jnp.float32),
              jax.ShapeDtypeStruct((N, F), jnp.float32)],
    mesh=_SC_MESH,
    scratch_types=[pltpu.VMEM((NCH, C), jnp.int32),
                   pltpu.VMEM((C, F), jnp.float32),
                   pltpu.VMEM((C, F), jnp.float32),
                   pltpu.VMEM_SHARED((N, F), jnp.float32),
                   pltpu.SemaphoreType.DMA,
                   pltpu.SemaphoreType.DMA],
)(_scatter_body)


BE1 = 1600
BE2 = 1000
BN = 2000
Z = 2 * F + DPE


def _sig(v):
    return 0.5 + 0.5 * jnp.tanh(0.5 * v)


def _p1_body(e_ref, xd_ref, xs_ref, Wpre_ref, bpre_ref,
             Wcat_ref, bcat_ref, Wcd_ref,
             msg_ref, c_ref, d_ref):
    bf = jnp.bfloat16
    e0 = jnp.tanh(
        jnp.dot(e_ref[...].astype(bf), Wpre_ref[...],
                preferred_element_type=jnp.float32) + bpre_ref[...])
    z = jnp.concatenate([xd_ref[...].astype(bf), xs_ref[...].astype(bf),
                         e0.astype(bf)], axis=1)
    u = (jnp.dot(z, Wcat_ref[...], preferred_element_type=jnp.float32)
         + bcat_ref[...])
    msg_ref[...] = jax.nn.relu(u[:, :F]) * _sig(u[:, F:2 * F])
    e1 = (e0 * (1.0 + _sig(u[:, 2 * F:]))).astype(bf)
    cd = jnp.dot(e1, Wcd_ref[...], preferred_element_type=jnp.float32)
    c_ref[...] = cd[:, :F].astype(bf)
    d_ref[...] = cd[:, F:].astype(bf)


def _p1(e, xd, xs, W_pre, b_pre, Wcat, bcat, Wcd):
    full = lambda shp: pl.BlockSpec(shp, lambda i: (0, 0))
    eb = lambda w: pl.BlockSpec((BE1, w), lambda i: (i, 0))
    return pl.pallas_call(
        _p1_body,
        grid=(EH // BE1,),
        in_specs=[eb(DE), eb(F), eb(F),
                  full((DE, DPE)), full((1, DPE)),
                  full((Z, Z)), full((1, Z)),
                  full((DPE, 2 * F))],
        out_specs=[eb(F), eb(F), eb(F)],
        out_shape=[jax.ShapeDtypeStruct((EH, F), jnp.float32),
                   jax.ShapeDtypeStruct((EH, F), jnp.bfloat16),
                   jax.ShapeDtypeStruct((EH, F), jnp.bfloat16)],
    )(e, xd, xs, W_pre, b_pre, Wcat, bcat, Wcd)


def _add_body(p0_ref, p1_ref, x_ref, x1_ref, sum_ref):
    i = pl.program_id(0)
    v = p0_ref[...] + p1_ref[...] - x_ref[...]
    x1_ref[...] = v

    @pl.when(i == 0)
    def _():
        sum_ref[...] = jnp.zeros_like(sum_ref)

    sum_ref[...] += jnp.sum(v, axis=0, keepdims=True)


def _addsum(p0, p1, x):
    return pl.pallas_call(
        _add_body,
        grid=(N // BN,),
        in_specs=[pl.BlockSpec((BN, F), lambda i: (i, 0)),
                  pl.BlockSpec((BN, F), lambda i: (i, 0)),
                  pl.BlockSpec((BN, F), lambda i: (i, 0))],
        out_specs=[pl.BlockSpec((BN, F), lambda i: (i, 0)),
                   pl.BlockSpec((1, F), lambda i: (0, 0))],
        out_shape=[jax.ShapeDtypeStruct((N, F), jnp.float32),
                   jax.ShapeDtypeStruct((1, F), jnp.float32)],
    )(p0, p1, x)


def _p2_body(x1d_ref, x1s_ref, c_ref, d_ref,
             Ws2n_ref, bs2_ref, Wf2n_ref, bf2_ref, acc_ref):
    i = pl.program_id(0)

    @pl.when(i == 0)
    def _():
        acc_ref[...] = jnp.zeros_like(acc_ref)

    z2 = jnp.concatenate([x1d_ref[...].astype(jnp.bfloat16),
                          x1s_ref[...].astype(jnp.bfloat16)], axis=1)
    s2 = (jnp.dot(z2, Ws2n_ref[...], preferred_element_type=jnp.float32)
          + c_ref[...] + bs2_ref[...])
    f2 = (jnp.dot(z2, Wf2n_ref[...], preferred_element_type=jnp.float32)
          + d_ref[...] + bf2_ref[...])
    m = jax.nn.relu(s2) * _sig(f2)
    acc_ref[...] += jnp.sum(m, axis=0, keepdims=True)


def _p2(x1d, x1s, cc, dd, Ws2n, bs2, Wf2n, bf2):
    full = lambda shp: pl.BlockSpec(shp, lambda i: (0, 0))
    eb = lambda w: pl.BlockSpec((BE2, w), lambda i: (i, 0))
    return pl.pallas_call(
        _p2_body,
        grid=(EH // BE2,),
        in_specs=[eb(F), eb(F), eb(F), eb(F),
                  full((2 * F, F)), full((1, F)),
                  full((2 * F, F)), full((1, F))],
        out_specs=full((1, F)),
        out_shape=jax.ShapeDtypeStruct((1, F), jnp.float32),
    )(x1d, x1s, cc, dd, Ws2n, bs2, Wf2n, bf2)


def _finish_body(sumx1_ref, acca_ref, accb_ref, Wout_ref, bout_ref, out_ref):
    pooled = sumx1_ref[...] + acca_ref[...] + accb_ref[...]
    logits = (jnp.dot(pooled, Wout_ref[...],
                      preferred_element_type=jnp.float32) + bout_ref[...])
    mx = jnp.max(logits, axis=-1, keepdims=True)
    ex = jnp.exp(logits - mx)
    out_ref[...] = ex / jnp.sum(ex, axis=-1, keepdims=True)


def _finish(sumx1, acca, accb, W_out, b_out):
    return pl.pallas_call(
        _finish_body,
        out_shape=jax.ShapeDtypeStruct((1, 32), jnp.float32),
    )(sumx1, acca, accb, W_out, b_out)


def kernel(x, edge_index, e,
           W_pre, b_pre,
           Wf1, bf1, Ws1, bs1, We1, be1,
           Wf2, bf2, Ws2, bs2, We2, be2,
           W_out, b_out):
    dst = edge_index[0]
    src = edge_index[1]

    bf = jnp.bfloat16
    dsts = (dst[:EH], dst[EH:])
    srcs = (src[:EH], src[EH:])
    es = (e[:EH], e[EH:])
    Wpre16 = W_pre.astype(bf)
    bpre = b_pre.reshape(1, DPE)

    Wcat = jnp.concatenate([Ws1, Wf1, We1], axis=1).astype(bf)
    bcat = jnp.concatenate([bs1, bf1, be1]).reshape(1, Z)
    Wcd = jnp.concatenate([Ws2[2 * F:], Wf2[2 * F:]], axis=1).astype(bf)

    g1 = [_gather2(x, dsts[h], srcs[h]) for h in range(2)]
    r1 = [_p1(es[h], g1[h][0], g1[h][1], Wpre16, bpre, Wcat, bcat, Wcd)
          for h in range(2)]
    p0, p1 = _scatter(r1[0][0], r1[1][0], x, dst.reshape(NW, NCH, C))
    x1, sumx1 = _addsum(p0, p1, x)
    g2 = [_gather2(x1, dsts[h], srcs[h]) for h in range(2)]
    accs = [_p2(g2[h][0], g2[h][1], r1[h][1], r1[h][2],
                Ws2[:2 * F].astype(bf), bs2.reshape(1, F),
                Wf2[:2 * F].astype(bf), bf2.reshape(1, F))
            for h in range(2)]
    out = _finish(sumx1, accs[0], accs[1], W_out, b_out.reshape(1, 32))
    return out.reshape(32)


# speedup vs baseline: 3.5293x; 1.0701x over previous
import functools

import jax
import jax.numpy as jnp
from jax import lax
from jax.experimental import pallas as pl
from jax.experimental.pallas import tpu as pltpu
from jax.experimental.pallas import tpu_sc as plsc

N = 10000
E = 160000
F = 128
DE = 16
DPE = 768

NC = 2
NS = 16
NW = NC * NS
PER_W = E // NW
C = 40
G_ITERS = PER_W // C
RT = 640
RT_LAST = N - 15 * RT

_SC_MESH = plsc.VectorSubcoreMesh(core_axis_name="c", subcore_axis_name="s")


CG = 128
EH = E // 2
CH_H = EH // CG
T_FULL = CH_H // NW
W_EXTRA = CH_H - T_FULL * NW


def _gather_body(tab_hbm, dst_hbm, src_hbm, xd_hbm, xs_hbm,
                 idxa, idxb, rda, rdb, rsa, rsb,
                 gda, gdb, gsa, gsb):
    c = lax.axis_index("c")
    s = lax.axis_index("s")
    w = c * NS + s

    def fire(t, idx, rd, rs, gd, gs):
        off = pl.multiple_of((w + t * NW) * CG, 8)
        pltpu.sync_copy(dst_hbm.at[pl.ds(off, CG)], idx.at[0])
        pltpu.sync_copy(src_hbm.at[pl.ds(off, CG)], idx.at[1])
        pltpu.make_async_copy(tab_hbm.at[idx.at[0]], rd, gd).start()
        pltpu.make_async_copy(tab_hbm.at[idx.at[1]], rs, gs).start()

    def drain(t, idx, rd, rs, gd, gs):
        off = pl.multiple_of((w + t * NW) * CG, 8)
        pltpu.make_async_copy(tab_hbm.at[idx.at[0]], rd, gd).wait()
        pltpu.sync_copy(rd, xd_hbm.at[pl.ds(off, CG)])
        pltpu.make_async_copy(tab_hbm.at[idx.at[1]], rs, gs).wait()
        pltpu.sync_copy(rs, xs_hbm.at[pl.ds(off, CG)])

    fire(0, idxa, rda, rsa, gda, gsa)

    def body(k, carry):
        fire(2 * k + 1, idxb, rdb, rsb, gdb, gsb)
        drain(2 * k, idxa, rda, rsa, gda, gsa)
        fire(2 * k + 2, idxa, rda, rsa, gda, gsa)
        drain(2 * k + 1, idxb, rdb, rsb, gdb, gsb)
        return carry

    lax.fori_loop(0, (T_FULL - 1) // 2, body, 0)
    drain(T_FULL - 1, idxa, rda, rsa, gda, gsa)

    @pl.when(w < W_EXTRA)
    def _():
        fire(T_FULL, idxb, rdb, rsb, gdb, gsb)
        drain(T_FULL, idxb, rdb, rsb, gdb, gsb)


_gather2 = functools.partial(
    pl.kernel,
    out_type=[jax.ShapeDtypeStruct((EH, F), jnp.float32),
              jax.ShapeDtypeStruct((EH, F), jnp.float32)],
    mesh=_SC_MESH,
    scratch_types=[pltpu.VMEM((2, CG), jnp.int32),
                   pltpu.VMEM((2, CG), jnp.int32),
                   pltpu.VMEM((CG, F), jnp.float32),
                   pltpu.VMEM((CG, F), jnp.float32),
                   pltpu.VMEM((CG, F), jnp.float32),
                   pltpu.VMEM((CG, F), jnp.float32),
                   pltpu.SemaphoreType.DMA,
                   pltpu.SemaphoreType.DMA,
                   pltpu.SemaphoreType.DMA,
                   pltpu.SemaphoreType.DMA],
)(_gather_body)


CH_S = EH // C
TS_FULL = CH_S // NW
WS_EXTRA = CH_S - TS_FULL * NW


def _scatter_body(msg_hbm, init0_hbm, init1_hbm, dst_hbm,
                  p0_hbm, p1_hbm,
                  idxa, idxb, ra, rb, acc_sh, ia, ib, ma, mb):
    c = lax.axis_index("c")
    s = lax.axis_index("s")
    w = c * NS + s
    roff = s * RT

    @pl.when((c == 0) & (s < NS - 1))
    def _():
        pltpu.sync_copy(init0_hbm.at[pl.ds(roff, RT)],
                        acc_sh.at[pl.ds(roff, RT)])

    @pl.when((c == 0) & (s == NS - 1))
    def _():
        pltpu.sync_copy(init0_hbm.at[pl.ds(roff, RT_LAST)],
                        acc_sh.at[pl.ds(roff, RT_LAST)])

    @pl.when((c != 0) & (s < NS - 1))
    def _():
        pltpu.sync_copy(init1_hbm.at[pl.ds(roff, RT)],
                        acc_sh.at[pl.ds(roff, RT)])

    @pl.when((c != 0) & (s == NS - 1))
    def _():
        pltpu.sync_copy(init1_hbm.at[pl.ds(roff, RT_LAST)],
                        acc_sh.at[pl.ds(roff, RT_LAST)])

    plsc.subcore_barrier()

    def fire(t, idx, rbuf, si, sm):
        off = pl.multiple_of((w + t * NW) * C, 8)
        pltpu.make_async_copy(dst_hbm.at[pl.ds(off, C)], idx, si).start()
        pltpu.make_async_copy(msg_hbm.at[pl.ds(off, C)], rbuf, sm).start()

    def drain(t, idx, rbuf, si, sm):
        pltpu.make_async_copy(dst_hbm.at[pl.ds(0, C)], idx, si).wait()
        pltpu.make_async_copy(msg_hbm.at[pl.ds(0, C)], rbuf, sm).wait()
        pltpu.sync_copy(rbuf, acc_sh.at[idx], add=True)

    fire(0, idxa, ra, ia, ma)

    def body(k, carry):
        fire(2 * k + 1, idxb, rb, ib, mb)
        drain(2 * k, idxa, ra, ia, ma)
        fire(2 * k + 2, idxa, ra, ia, ma)
        drain(2 * k + 1, idxb, rb, ib, mb)
        return carry

    lax.fori_loop(0, TS_FULL // 2 - 1, body, 0)
    fire(TS_FULL - 1, idxb, rb, ib, mb)
    drain(TS_FULL - 2, idxa, ra, ia, ma)
    drain(TS_FULL - 1, idxb, rb, ib, mb)

    @pl.when(w < WS_EXTRA)
    def _():
        fire(TS_FULL, idxa, ra, ia, ma)
        drain(TS_FULL, idxa, ra, ia, ma)

    plsc.subcore_barrier()

    @pl.when((c == 0) & (s < NS - 1))
    def _():
        pltpu.sync_copy(acc_sh.at[pl.ds(roff, RT)], p0_hbm.at[pl.ds(roff, RT)])

    @pl.when((c == 0) & (s == NS - 1))
    def _():
        pltpu.sync_copy(acc_sh.at[pl.ds(roff, RT_LAST)],
                        p0_hbm.at[pl.ds(roff, RT_LAST)])

    @pl.when((c != 0) & (s < NS - 1))
    def _():
        pltpu.sync_copy(acc_sh.at[pl.ds(roff, RT)], p1_hbm.at[pl.ds(roff, RT)])

    @pl.when((c != 0) & (s == NS - 1))
    def _():
        pltpu.sync_copy(acc_sh.at[pl.ds(roff, RT_LAST)],
                        p1_hbm.at[pl.ds(roff, RT_LAST)])


_scatter = functools.partial(
    pl.kernel,
    out_type=[jax.ShapeDtypeStruct((N, F), jnp.float32),
              jax.ShapeDtypeStruct((N, F), jnp.float32)],
    mesh=_SC_MESH,
    scratch_types=[pltpu.VMEM((C,), jnp.int32),
                   pltpu.VMEM((C,), jnp.int32),
                   pltpu.VMEM((C, F), jnp.float32),
                   pltpu.VMEM((C, F), jnp.float32),
                   pltpu.VMEM_SHARED((N, F), jnp.float32),
                   pltpu.SemaphoreType.DMA,
                   pltpu.SemaphoreType.DMA,
                   pltpu.SemaphoreType.DMA,
                   pltpu.SemaphoreType.DMA],
)(_scatter_body)


BE1 = 1600
BE2 = 1000
BN = 2000
Z = 2 * F + DPE


def _sig(v):
    return 0.5 + 0.5 * jnp.tanh(0.5 * v)


def _p1_body(e_ref, xd_ref, xs_ref, Wpre_ref, bpre_ref,
             Wcat_ref, bcat_ref, Wcd_ref,
             msg_ref, c_ref, d_ref):
    bf = jnp.bfloat16
    e0 = jnp.tanh(
        jnp.dot(e_ref[...].astype(bf), Wpre_ref[...],
                preferred_element_type=jnp.float32) + bpre_ref[...])
    z = jnp.concatenate([xd_ref[...].astype(bf), xs_ref[...].astype(bf),
                         e0.astype(bf)], axis=1)
    u = (jnp.dot(z, Wcat_ref[...], preferred_element_type=jnp.float32)
         + bcat_ref[...])
    msg_ref[...] = jax.nn.relu(u[:, :F]) * _sig(u[:, F:2 * F])
    e1 = (e0 * (1.0 + _sig(u[:, 2 * F:]))).astype(bf)
    cd = jnp.dot(e1, Wcd_ref[...], preferred_element_type=jnp.float32)
    c_ref[...] = cd[:, :F].astype(bf)
    d_ref[...] = cd[:, F:].astype(bf)


def _p1(e, xd, xs, W_pre, b_pre, Wcat, bcat, Wcd):
    full = lambda shp: pl.BlockSpec(shp, lambda i: (0, 0))
    eb = lambda w: pl.BlockSpec((BE1, w), lambda i: (i, 0))
    return pl.pallas_call(
        _p1_body,
        grid=(EH // BE1,),
        in_specs=[eb(DE), eb(F), eb(F),
                  full((DE, DPE)), full((1, DPE)),
                  full((Z, Z)), full((1, Z)),
                  full((DPE, 2 * F))],
        out_specs=[eb(F), eb(F), eb(F)],
        out_shape=[jax.ShapeDtypeStruct((EH, F), jnp.float32),
                   jax.ShapeDtypeStruct((EH, F), jnp.bfloat16),
                   jax.ShapeDtypeStruct((EH, F), jnp.bfloat16)],
    )(e, xd, xs, W_pre, b_pre, Wcat, bcat, Wcd)


def _add_body(p0_ref, p1_ref, x_ref, x1_ref, sum_ref):
    i = pl.program_id(0)
    v = p0_ref[...] + p1_ref[...] - x_ref[...]
    x1_ref[...] = v

    @pl.when(i == 0)
    def _():
        sum_ref[...] = jnp.zeros_like(sum_ref)

    sum_ref[...] += jnp.sum(v, axis=0, keepdims=True)


def _addsum(p0, p1, x):
    return pl.pallas_call(
        _add_body,
        grid=(N // BN,),
        in_specs=[pl.BlockSpec((BN, F), lambda i: (i, 0)),
                  pl.BlockSpec((BN, F), lambda i: (i, 0)),
                  pl.BlockSpec((BN, F), lambda i: (i, 0))],
        out_specs=[pl.BlockSpec((BN, F), lambda i: (i, 0)),
                   pl.BlockSpec((1, F), lambda i: (0, 0))],
        out_shape=[jax.ShapeDtypeStruct((N, F), jnp.float32),
                   jax.ShapeDtypeStruct((1, F), jnp.float32)],
    )(p0, p1, x)


def _p2_body(x1d_ref, x1s_ref, c_ref, d_ref,
             Ws2n_ref, bs2_ref, Wf2n_ref, bf2_ref, acc_ref):
    i = pl.program_id(0)

    @pl.when(i == 0)
    def _():
        acc_ref[...] = jnp.zeros_like(acc_ref)

    z2 = jnp.concatenate([x1d_ref[...].astype(jnp.bfloat16),
                          x1s_ref[...].astype(jnp.bfloat16)], axis=1)
    s2 = (jnp.dot(z2, Ws2n_ref[...], preferred_element_type=jnp.float32)
          + c_ref[...] + bs2_ref[...])
    f2 = (jnp.dot(z2, Wf2n_ref[...], preferred_element_type=jnp.float32)
          + d_ref[...] + bf2_ref[...])
    m = jax.nn.relu(s2) * _sig(f2)
    acc_ref[...] += jnp.sum(m, axis=0, keepdims=True)


def _p2(x1d, x1s, cc, dd, Ws2n, bs2, Wf2n, bf2):
    full = lambda shp: pl.BlockSpec(shp, lambda i: (0, 0))
    eb = lambda w: pl.BlockSpec((BE2, w), lambda i: (i, 0))
    return pl.pallas_call(
        _p2_body,
        grid=(EH // BE2,),
        in_specs=[eb(F), eb(F), eb(F), eb(F),
                  full((2 * F, F)), full((1, F)),
                  full((2 * F, F)), full((1, F))],
        out_specs=full((1, F)),
        out_shape=jax.ShapeDtypeStruct((1, F), jnp.float32),
    )(x1d, x1s, cc, dd, Ws2n, bs2, Wf2n, bf2)


def _finish_body(sumx1_ref, acca_ref, accb_ref, Wout_ref, bout_ref, out_ref):
    pooled = sumx1_ref[...] + acca_ref[...] + accb_ref[...]
    logits = (jnp.dot(pooled, Wout_ref[...],
                      preferred_element_type=jnp.float32) + bout_ref[...])
    mx = jnp.max(logits, axis=-1, keepdims=True)
    ex = jnp.exp(logits - mx)
    out_ref[...] = ex / jnp.sum(ex, axis=-1, keepdims=True)


def _finish(sumx1, acca, accb, W_out, b_out):
    return pl.pallas_call(
        _finish_body,
        out_shape=jax.ShapeDtypeStruct((1, 32), jnp.float32),
    )(sumx1, acca, accb, W_out, b_out)


def kernel(x, edge_index, e,
           W_pre, b_pre,
           Wf1, bf1, Ws1, bs1, We1, be1,
           Wf2, bf2, Ws2, bs2, We2, be2,
           W_out, b_out):
    dst = edge_index[0]
    src = edge_index[1]

    bf = jnp.bfloat16
    dsts = (dst[:EH], dst[EH:])
    srcs = (src[:EH], src[EH:])
    es = (e[:EH], e[EH:])
    Wpre16 = W_pre.astype(bf)
    bpre = b_pre.reshape(1, DPE)

    Wcat = jnp.concatenate([Ws1, Wf1, We1], axis=1).astype(bf)
    bcat = jnp.concatenate([bs1, bf1, be1]).reshape(1, Z)
    Wcd = jnp.concatenate([Ws2[2 * F:], Wf2[2 * F:]], axis=1).astype(bf)

    g1 = [_gather2(x, dsts[h], srcs[h]) for h in range(2)]
    r1 = [_p1(es[h], g1[h][0], g1[h][1], Wpre16, bpre, Wcat, bcat, Wcd)
          for h in range(2)]
    pa0, pa1 = _scatter(r1[0][0], x, x, dsts[0])
    p0, p1 = _scatter(r1[1][0], pa0, pa1, dsts[1])
    x1, sumx1 = _addsum(p0, p1, x)
    g2 = [_gather2(x1, dsts[h], srcs[h]) for h in range(2)]
    accs = [_p2(g2[h][0], g2[h][1], r1[h][1], r1[h][2],
                Ws2[:2 * F].astype(bf), bs2.reshape(1, F),
                Wf2[:2 * F].astype(bf), bf2.reshape(1, F))
            for h in range(2)]
    out = _finish(sumx1, accs[0], accs[1], W_out, b_out.reshape(1, 32))
    return out.reshape(32)
